# baseline (speedup 1.0000x reference)
import dataclasses
import functools

import jax
import jax.numpy as jnp
from jax import lax
from jax.experimental import pallas as pl
from jax.experimental.pallas import tpu as pltpu
from jax.experimental.pallas import tpu_sc as plsc

_N = 4096
_M = 8
_TOPK = 10
_NCAND = 16
_ROWS = 32
_COLS = 128
_L = 16
_NEG = -1.0
_BIGI = 1 << 30


def _tc_body(x_ref, mt_ref, prev_ref, change_ref, explained_ref,
             residual_ref, niche_ref):
    x = x_ref[...]
    prev = prev_ref[...]
    change = x - prev
    change_ref[...] = change

    niche_list = []
    explained = jnp.zeros((_ROWS, _COLS), jnp.float32)
    for j in range(_M):
        niche_list.append(jnp.sum(mt_ref[j] * change))
    for j in range(_M):
        explained = explained + niche_list[j] * mt_ref[j]
    explained_ref[...] = explained
    residual_ref[...] = change - explained
    niche_ref[0, :] = jnp.stack(niche_list)


def _run_tc(x2, mt3, prev2):
    return pl.pallas_call(
        _tc_body,
        out_shape=(
            jax.ShapeDtypeStruct((_ROWS, _COLS), jnp.float32),
            jax.ShapeDtypeStruct((_ROWS, _COLS), jnp.float32),
            jax.ShapeDtypeStruct((_ROWS, _COLS), jnp.float32),
            jax.ShapeDtypeStruct((1, _M), jnp.float32),
        ),
    )(x2, mt3, prev2)


def _sc_topk_body(x_hbm, prev_hbm, out_hbm, xv, pv, tv_ref, ti_ref, ob, sem):
    cid = lax.axis_index("c")
    sid = lax.axis_index("s")

    @pl.when(jnp.logical_and(cid == 0, sid == 0))
    def _():
        pltpu.async_copy(x_hbm, xv, sem).wait()
        pltpu.async_copy(prev_hbm, pv, sem).wait()
        lanes = lax.iota(jnp.int32, _L)

        def step(i, carry):
            tvals, tidx = carry
            base = i * _L
            v = jnp.abs(xv[pl.ds(base, _L)] - pv[pl.ds(base, _L)])
            sv, si = plsc.sort_key_val(v, lanes + base, descending=True)
            take = sv > tvals
            nv = jnp.where(take, sv, tvals)
            ni = jnp.where(take, si, tidx)
            nv2, ni2 = plsc.sort_key_val(nv, ni)
            return (nv2, ni2)

        init = (jnp.full((_L,), _NEG, jnp.float32), jnp.zeros((_L,), jnp.int32))
        tvals, tidx = lax.fori_loop(0, _N // _L, step, init)
        tv_ref[...] = tvals
        ti_ref[...] = tidx

        prods = []
        keys = []
        keybase = tidx * _N
        for s in range(1, _L):
            perm = (lanes + s) & (_L - 1)
            rv = plsc.load_gather(tv_ref, [perm])
            ri = plsc.load_gather(ti_ref, [perm])
            prods.append(tvals * rv)
            keys.append(keybase + ri)

        selv = jnp.zeros((_L,), jnp.float32)
        selk = jnp.zeros((_L,), jnp.int32)
        first_m = None
        for t in range(_TOPK):
            m = prods[0]
            for s in range(1, _L - 1):
                m = jnp.maximum(m, prods[s])
            ms = jnp.max(m)
            if first_m is None:
                first_m = ms
            msv = jnp.full((_L,), ms)
            kc = jnp.full((_L,), _BIGI, jnp.int32)
            for s in range(_L - 1):
                kc = jnp.minimum(kc, jnp.where(prods[s] == msv, keys[s], _BIGI))
            km = jnp.min(kc)
            kmv = jnp.full((_L,), km)
            for s in range(_L - 1):
                prods[s] = jnp.where(keys[s] == kmv, _NEG, prods[s])
            selv = jnp.where(lanes == t, msv, selv)
            selk = jnp.where(lanes == t, kmv, selk)

        inv_max = jnp.full((_L,), jnp.maximum(first_m, 1e-8))
        ob[pl.ds(0, _L)] = (selk >> 12).astype(jnp.float32)
        ob[pl.ds(_L, _L)] = (selk & (_N - 1)).astype(jnp.float32)
        ob[pl.ds(2 * _L, _L)] = selv / inv_max
        pltpu.sync_copy(ob, out_hbm)


@functools.cache
def _sc_topk_kernel():
    mesh = plsc.VectorSubcoreMesh(core_axis_name="c", subcore_axis_name="s")
    cp = pltpu.CompilerParams()
    if "needs_layout_passes" in pltpu.CompilerParams.__dataclass_fields__:
        cp = dataclasses.replace(cp, needs_layout_passes=False)
    return pl.kernel(
        _sc_topk_body,
        mesh=mesh,
        compiler_params=cp,
        out_type=jax.ShapeDtypeStruct((3 * _L,), jnp.float32),
        scratch_types=[
            pltpu.VMEM((_N,), jnp.float32),
            pltpu.VMEM((_N,), jnp.float32),
            pltpu.VMEM((_L,), jnp.float32),
            pltpu.VMEM((_L,), jnp.int32),
            pltpu.VMEM((3 * _L,), jnp.float32),
            pltpu.SemaphoreType.DMA,
        ],
    )


@jax.jit
def kernel(current_neuron_state, mechanism_state, prev_state):
    x2 = current_neuron_state.reshape(_ROWS, _COLS)
    prev2 = prev_state.reshape(_ROWS, _COLS)
    mt3 = mechanism_state.T.reshape(_M, _ROWS, _COLS)
    change, explained, residual, niche = _run_tc(x2, mt3, prev2)
    sc = _sc_topk_kernel()(current_neuron_state, prev_state)
    trip = jnp.stack(
        [sc[0:_TOPK], sc[_L:_L + _TOPK], sc[2 * _L:2 * _L + _TOPK]], axis=1
    ).reshape(-1)
    return jnp.concatenate([
        change.reshape(-1), explained.reshape(-1), residual.reshape(-1),
        trip, niche.reshape(-1),
    ])


# speedup vs baseline: 1.8171x; 1.8171x over previous
import jax
import jax.numpy as jnp
from jax import lax
from jax.experimental import pallas as pl
from jax.experimental.pallas import tpu as pltpu

_N = 4096
_M = 8
_TOPK = 10
_NCAND = 16
_NEG = -1.0
_BIGI = 1 << 30
_OUT = 3 * _N + 3 * _TOPK + _M


def _tc_body(x_ref, m_ref, prev_ref, out_ref):
    x = x_ref[...]
    prev = prev_ref[...]
    mm = m_ref[...]
    change = x - prev
    niche = lax.dot_general(change, mm, (((1,), (0,)), ((), ())),
                            preferred_element_type=jnp.float32)
    explained = lax.dot_general(niche, mm, (((1,), (1,)), ((), ())),
                                preferred_element_type=jnp.float32)
    out_ref[:, pl.ds(0, _N)] = change
    out_ref[:, pl.ds(_N, _N)] = explained
    out_ref[:, pl.ds(2 * _N, _N)] = change - explained

    a = jnp.abs(change)
    fidx = lax.broadcasted_iota(jnp.int32, (1, _N), 1)
    vals = []
    idxs = []
    for _ in range(_NCAND):
        mx = jnp.max(a)
        i = jnp.min(jnp.where(a == mx, fidx, _BIGI))
        vals.append(mx)
        idxs.append(i)
        a = jnp.where(fidx == i, _NEG, a)

    v16 = jnp.stack(vals)
    i16 = jnp.stack(idxs)
    prod = v16[:, None] * v16[None, :]
    keys = i16[:, None] * _N + i16[None, :]
    rr = lax.broadcasted_iota(jnp.int32, (_NCAND, _NCAND), 0)
    cc = lax.broadcasted_iota(jnp.int32, (_NCAND, _NCAND), 1)
    prod = jnp.where(rr == cc, _NEG, prod)

    tail = []
    max_val = None
    for t in range(_TOPK):
        mx = jnp.max(prod)
        k = jnp.min(jnp.where(prod == mx, keys, _BIGI))
        prod = jnp.where(keys == k, _NEG, prod)
        if max_val is None:
            max_val = jnp.maximum(mx, 1e-8)
        tail.append((k >> 12).astype(jnp.float32))
        tail.append((k & (_N - 1)).astype(jnp.float32))
        tail.append(mx / max_val)
    for j in range(_M):
        tail.append(niche[0, j])
    out_ref[:, pl.ds(3 * _N, 3 * _TOPK + _M)] = jnp.stack(tail)[None, :]


@jax.jit
def kernel(current_neuron_state, mechanism_state, prev_state):
    out = pl.pallas_call(
        _tc_body,
        out_shape=jax.ShapeDtypeStruct((1, _OUT), jnp.float32),
    )(current_neuron_state.reshape(1, _N), mechanism_state,
      prev_state.reshape(1, _N))
    return out.reshape(_OUT)
